# baseline (speedup 1.0000x reference)
import functools

import jax
import jax.numpy as jnp
from jax import lax
from jax.experimental import pallas as pl
from jax.experimental.pallas import tpu as pltpu
from jax.experimental.pallas import tpu_sc as plsc

N = 10000
E = 160000
DF = 256
DC = 64

NC = 2
NT = 16
CHUNK = 128
EPT = 10240
EPAD = NT * EPT
NCH = EPT // CHUNK
EPT2 = EPAD // (NC * NT)
NCH2 = EPT2 // CHUNK

RACC = 10112
RPT = RACC // NT
OPT = 624

NB = 2

_MESH = plsc.VectorSubcoreMesh(core_axis_name="c", subcore_axis_name="s")


def _copy_out(accum, out_hbm, cid, tid):
    pltpu.sync_copy(accum.at[pl.ds(tid * OPT, OPT)],
                    out_hbm.at[pl.ds(cid * N + tid * OPT, OPT)])

    @pl.when(tid == NT - 1)
    def _tail():
        pltpu.sync_copy(accum.at[pl.ds(NT * OPT, N - NT * OPT)],
                        out_hbm.at[pl.ds(cid * N + NT * OPT, N - NT * OPT)])


BANK = 40


GSPLIT = 1
SUB = CHUNK // GSPLIT


def _make_seg_sum(nch, src_rows_fn, dst_rows_fn, col_fn=None):
    nbanks = nch // BANK

    @functools.partial(
        pl.kernel,
        out_type=jax.ShapeDtypeStruct((2 * N, 128), jnp.float32),
        mesh=_MESH,
        scratch_types=[
            pltpu.VMEM_SHARED((RACC, 128), jnp.float32),
            pltpu.VMEM((BANK, CHUNK), jnp.int32),
            pltpu.VMEM((BANK, CHUNK), jnp.int32),
            pltpu.VMEM((NB, CHUNK, 128), jnp.float32),
            [[pltpu.SemaphoreType.DMA] * GSPLIT] * NB,
            [pltpu.SemaphoreType.DMA] * NB,
        ],
    )
    def seg(rows_hbm, src_hbm, dst_hbm, init_hbm, out_hbm,
            accum, sbank, dbank, rows_v, gsems, ssems):
        cid = lax.axis_index("c")
        tid = lax.axis_index("s")
        pltpu.sync_copy(init_hbm.at[pl.ds(cid * RPT, RPT)],
                        accum.at[pl.ds(tid * RPT, RPT)])
        plsc.subcore_barrier()

        def g_copy(c, b, h):
            idx = sbank.at[c, pl.ds(h * SUB, SUB)]
            src = (rows_hbm.at[idx] if col_fn is None
                   else rows_hbm.at[idx, pl.ds(col_fn(cid), 128)])
            return pltpu.make_async_copy(
                src, rows_v.at[b, pl.ds(h * SUB, SUB)], gsems[b][h])

        def g_start(c, b):
            for h in range(GSPLIT):
                g_copy(c, b, h).start()

        def g_wait(c, b):
            for h in range(GSPLIT):
                g_copy(c, b, h).wait()

        def s_copy(c, b):
            return pltpu.make_async_copy(
                rows_v.at[b], accum.at[dbank.at[c]], ssems[b])

        for k in range(nbanks):
            pltpu.sync_copy(
                src_hbm.at[pl.ds(src_rows_fn(cid, tid) + k * BANK, BANK)],
                sbank)
            pltpu.sync_copy(
                dst_hbm.at[pl.ds(dst_rows_fn(cid, tid) + k * BANK, BANK)],
                dbank)
            g_start(0, 0)

            def pair(p, carry):
                for b in range(NB):
                    c = p * NB + b

                    @pl.when(c + 1 < BANK)
                    def _prefetch():
                        @pl.when(c >= 1)
                        def _drain():
                            s_copy(c - 1, 1 - b).wait()

                        g_start(c + 1, 1 - b)

                    g_wait(c, b)
                    s_copy(c, b).start(add=True)
                return carry

            lax.fori_loop(0, BANK // NB, pair, 0)
            s_copy(BANK - 2, 0).wait()
            s_copy(BANK - 1, 1).wait()

        plsc.subcore_barrier()
        _copy_out(accum, out_hbm, cid, tid)

    return seg


_seg_sum_l1 = _make_seg_sum(
    NCH,
    lambda cid, tid: tid * NCH,
    lambda cid, tid: tid * NCH,
    col_fn=lambda cid: cid * 128,
)


def _edge_rows(cid, tid):
    return cid * (EPAD // (2 * CHUNK)) + tid * NCH2


_seg_sum_l2 = _make_seg_sum(NCH2, _edge_rows, _edge_rows)


def _tc_mm_body(a_ref, b_ref, w1a_ref, w1b_ref, b1_ref, w2_ref, p_ref):
    h = jnp.dot(a_ref[...], w1a_ref[...], preferred_element_type=jnp.float32)
    h = h + jnp.dot(b_ref[...], w1b_ref[...],
                    preferred_element_type=jnp.float32)
    h = jnp.maximum(h + b1_ref[...], 0.0)
    p = jnp.dot(h, w2_ref[...], preferred_element_type=jnp.float32)
    p_ref[...] = jnp.pad(p, ((0, 0), (0, 128 - DC)))


def _tc_dense(agg, W1a, W1b, b1, W2):
    BM = 1000
    return pl.pallas_call(
        _tc_mm_body,
        grid=(N // BM,),
        in_specs=[
            pl.BlockSpec((BM, 128), lambda i: (i, 0)),
            pl.BlockSpec((BM, 128), lambda i: (N // BM + i, 0)),
            pl.BlockSpec((128, DF), lambda i: (0, 0)),
            pl.BlockSpec((128, DF), lambda i: (0, 0)),
            pl.BlockSpec((1, DF), lambda i: (0, 0)),
            pl.BlockSpec((DF, DC), lambda i: (0, 0)),
        ],
        out_specs=pl.BlockSpec((BM, 128), lambda i: (i, 0)),
        out_shape=jax.ShapeDtypeStruct((N, 128), jnp.float32),
        compiler_params=pltpu.CompilerParams(
            dimension_semantics=("arbitrary",)),
    )(agg, agg, W1a, W1b, b1, W2)


def _tc_comb_body(a_ref, b_ref, o_ref):
    o_ref[...] = a_ref[:, :DC] + b_ref[:, :DC]


def _tc_combine(out2):
    BM = 1000
    return pl.pallas_call(
        _tc_comb_body,
        grid=(N // BM,),
        in_specs=[
            pl.BlockSpec((BM, 128), lambda i: (i, 0)),
            pl.BlockSpec((BM, 128), lambda i: (N // BM + i, 0)),
        ],
        out_specs=pl.BlockSpec((BM, DC), lambda i: (i, 0)),
        out_shape=jax.ShapeDtypeStruct((N, DC), jnp.float32),
        compiler_params=pltpu.CompilerParams(
            dimension_semantics=("arbitrary",)),
    )(out2, out2)


def kernel(x, adj, W1, b1, W2, b2):
    src = adj[0].astype(jnp.int32)
    dst = adj[1].astype(jnp.int32)
    pad = EPAD - E
    pad_i = jnp.arange(pad, dtype=jnp.int32)
    src_p = jnp.concatenate([src, pad_i % N])
    dst_p = jnp.concatenate([dst, N + pad_i % (RACC - N)])
    src2d = src_p.reshape(-1, CHUNK)
    dst2d = dst_p.reshape(-1, CHUNK)

    zinit = jnp.zeros((2 * RPT, 128), jnp.float32)
    agg = _seg_sum_l1(x, src2d, dst2d, zinit)

    p128 = _tc_dense(agg, W1[:128], W1[128:], b1.reshape(1, DF), W2)

    binit = jnp.concatenate([
        jnp.broadcast_to(jnp.pad(b2, (0, 128 - DC)), (RPT, 128)),
        jnp.zeros((RPT, 128), jnp.float32),
    ])
    out2 = _seg_sum_l2(p128, src2d, dst2d, binit)
    return _tc_combine(out2)


# speedup vs baseline: 1.0175x; 1.0175x over previous
import functools

import jax
import jax.numpy as jnp
from jax import lax
from jax.experimental import pallas as pl
from jax.experimental.pallas import tpu as pltpu
from jax.experimental.pallas import tpu_sc as plsc

N = 10000
E = 160000
DF = 256
DC = 64

NC = 2
NT = 16
CHUNK = 128
EPT = 10240
EPAD = NT * EPT
NCH = EPT // CHUNK
EPT2 = EPAD // (NC * NT)
NCH2 = EPT2 // CHUNK

RACC = 10112
RPT = RACC // NT
OPT = 624

NB = 2

_MESH = plsc.VectorSubcoreMesh(core_axis_name="c", subcore_axis_name="s")


def _copy_out(accum, out_hbm, cid, tid):
    pltpu.sync_copy(accum.at[pl.ds(tid * OPT, OPT)],
                    out_hbm.at[pl.ds(cid * N + tid * OPT, OPT)])

    @pl.when(tid == NT - 1)
    def _tail():
        pltpu.sync_copy(accum.at[pl.ds(NT * OPT, N - NT * OPT)],
                        out_hbm.at[pl.ds(cid * N + NT * OPT, N - NT * OPT)])


BANK = 40


GSPLIT = 4
SUB = CHUNK // GSPLIT


def _make_seg_sum(nch, src_rows_fn, dst_rows_fn, col_fn=None):
    nbanks = nch // BANK

    @functools.partial(
        pl.kernel,
        out_type=jax.ShapeDtypeStruct((2 * N, 128), jnp.float32),
        mesh=_MESH,
        scratch_types=[
            pltpu.VMEM_SHARED((RACC, 128), jnp.float32),
            pltpu.VMEM((BANK, CHUNK), jnp.int32),
            pltpu.VMEM((BANK, CHUNK), jnp.int32),
            pltpu.VMEM((NB, CHUNK, 128), jnp.float32),
            [[pltpu.SemaphoreType.DMA] * GSPLIT] * NB,
            [pltpu.SemaphoreType.DMA] * NB,
        ],
    )
    def seg(rows_hbm, src_hbm, dst_hbm, init_hbm, out_hbm,
            accum, sbank, dbank, rows_v, gsems, ssems):
        cid = lax.axis_index("c")
        tid = lax.axis_index("s")
        pltpu.sync_copy(init_hbm.at[pl.ds(cid * RPT, RPT)],
                        accum.at[pl.ds(tid * RPT, RPT)])
        plsc.subcore_barrier()

        def g_copy(c, b, h):
            idx = sbank.at[c, pl.ds(h * SUB, SUB)]
            src = (rows_hbm.at[idx] if col_fn is None
                   else rows_hbm.at[idx, pl.ds(col_fn(cid), 128)])
            return pltpu.make_async_copy(
                src, rows_v.at[b, pl.ds(h * SUB, SUB)], gsems[b][h])

        def g_start(c, b):
            for h in range(GSPLIT):
                g_copy(c, b, h).start()

        def g_wait(c, b):
            for h in range(GSPLIT):
                g_copy(c, b, h).wait()

        def s_copy(c, b):
            return pltpu.make_async_copy(
                rows_v.at[b], accum.at[dbank.at[c]], ssems[b])

        for k in range(nbanks):
            pltpu.sync_copy(
                src_hbm.at[pl.ds(src_rows_fn(cid, tid) + k * BANK, BANK)],
                sbank)
            pltpu.sync_copy(
                dst_hbm.at[pl.ds(dst_rows_fn(cid, tid) + k * BANK, BANK)],
                dbank)
            g_start(0, 0)

            def pair(p, carry):
                for b in range(NB):
                    c = p * NB + b

                    @pl.when(c + 1 < BANK)
                    def _prefetch():
                        @pl.when(c >= 1)
                        def _drain():
                            s_copy(c - 1, 1 - b).wait()

                        g_start(c + 1, 1 - b)

                    g_wait(c, b)
                    s_copy(c, b).start(add=True)
                return carry

            lax.fori_loop(0, BANK // NB, pair, 0)
            s_copy(BANK - 2, 0).wait()
            s_copy(BANK - 1, 1).wait()

        plsc.subcore_barrier()
        _copy_out(accum, out_hbm, cid, tid)

    return seg


_seg_sum_l1 = _make_seg_sum(
    NCH,
    lambda cid, tid: tid * NCH,
    lambda cid, tid: tid * NCH,
    col_fn=lambda cid: cid * 128,
)


def _edge_rows(cid, tid):
    return cid * (EPAD // (2 * CHUNK)) + tid * NCH2


_seg_sum_l2 = _make_seg_sum(NCH2, _edge_rows, _edge_rows)


def _tc_mm_body(a_ref, b_ref, w1a_ref, w1b_ref, b1_ref, w2_ref, p_ref):
    h = jnp.dot(a_ref[...], w1a_ref[...], preferred_element_type=jnp.float32)
    h = h + jnp.dot(b_ref[...], w1b_ref[...],
                    preferred_element_type=jnp.float32)
    h = jnp.maximum(h + b1_ref[...], 0.0)
    p = jnp.dot(h, w2_ref[...], preferred_element_type=jnp.float32)
    p_ref[...] = jnp.pad(p, ((0, 0), (0, 128 - DC)))


def _tc_dense(agg, W1a, W1b, b1, W2):
    BM = 2000
    return pl.pallas_call(
        _tc_mm_body,
        grid=(N // BM,),
        in_specs=[
            pl.BlockSpec((BM, 128), lambda i: (i, 0)),
            pl.BlockSpec((BM, 128), lambda i: (N // BM + i, 0)),
            pl.BlockSpec((128, DF), lambda i: (0, 0)),
            pl.BlockSpec((128, DF), lambda i: (0, 0)),
            pl.BlockSpec((1, DF), lambda i: (0, 0)),
            pl.BlockSpec((DF, DC), lambda i: (0, 0)),
        ],
        out_specs=pl.BlockSpec((BM, 128), lambda i: (i, 0)),
        out_shape=jax.ShapeDtypeStruct((N, 128), jnp.float32),
        compiler_params=pltpu.CompilerParams(
            dimension_semantics=("arbitrary",)),
    )(agg, agg, W1a, W1b, b1, W2)


def _tc_comb_body(a_ref, b_ref, o_ref):
    o_ref[...] = a_ref[:, :DC] + b_ref[:, :DC]


def _tc_combine(out2):
    BM = 1000
    return pl.pallas_call(
        _tc_comb_body,
        grid=(N // BM,),
        in_specs=[
            pl.BlockSpec((BM, 128), lambda i: (i, 0)),
            pl.BlockSpec((BM, 128), lambda i: (N // BM + i, 0)),
        ],
        out_specs=pl.BlockSpec((BM, DC), lambda i: (i, 0)),
        out_shape=jax.ShapeDtypeStruct((N, DC), jnp.float32),
        compiler_params=pltpu.CompilerParams(
            dimension_semantics=("arbitrary",)),
    )(out2, out2)


def kernel(x, adj, W1, b1, W2, b2):
    src = adj[0].astype(jnp.int32)
    dst = adj[1].astype(jnp.int32)
    pad = EPAD - E
    pad_i = jnp.arange(pad, dtype=jnp.int32)
    src_p = jnp.concatenate([src, pad_i % N])
    dst_p = jnp.concatenate([dst, N + pad_i % (RACC - N)])
    src2d = src_p.reshape(-1, CHUNK)
    dst2d = dst_p.reshape(-1, CHUNK)

    zinit = jnp.zeros((2 * RPT, 128), jnp.float32)
    agg = _seg_sum_l1(x, src2d, dst2d, zinit)

    p128 = _tc_dense(agg, W1[:128], W1[128:], b1.reshape(1, DF), W2)

    binit = jnp.concatenate([
        jnp.broadcast_to(jnp.pad(b2, (0, 128 - DC)), (RPT, 128)),
        jnp.zeros((RPT, 128), jnp.float32),
    ])
    out2 = _seg_sum_l2(p128, src2d, dst2d, binit)
    return _tc_combine(out2)


# speedup vs baseline: 1.0254x; 1.0077x over previous
import functools

import jax
import jax.numpy as jnp
from jax import lax
from jax.experimental import pallas as pl
from jax.experimental.pallas import tpu as pltpu
from jax.experimental.pallas import tpu_sc as plsc

N = 10000
E = 160000
DF = 256
DC = 64

NC = 2
NT = 16
CHUNK = 128
EPT = 10240
EPAD = NT * EPT
NCH = EPT // CHUNK
EPT2 = EPAD // (NC * NT)
NCH2 = EPT2 // CHUNK

RACC = 10112
RPT = RACC // NT
OPT = 624

NB = 2

_MESH = plsc.VectorSubcoreMesh(core_axis_name="c", subcore_axis_name="s")


def _copy_out(accum, out_hbm, cid, tid):
    pltpu.sync_copy(accum.at[pl.ds(tid * OPT, OPT)],
                    out_hbm.at[pl.ds(cid * N + tid * OPT, OPT)])

    @pl.when(tid == NT - 1)
    def _tail():
        pltpu.sync_copy(accum.at[pl.ds(NT * OPT, N - NT * OPT)],
                        out_hbm.at[pl.ds(cid * N + NT * OPT, N - NT * OPT)])


BANK = 40


GSPLIT = 4
SUB = CHUNK // GSPLIT


def _make_seg_sum(nch, src_rows_fn, dst_rows_fn, col_fn=None):
    nbanks = nch // BANK

    @functools.partial(
        pl.kernel,
        out_type=jax.ShapeDtypeStruct((2 * N, 128), jnp.float32),
        mesh=_MESH,
        scratch_types=[
            pltpu.VMEM_SHARED((RACC, 128), jnp.float32),
            pltpu.VMEM((BANK, CHUNK), jnp.int32),
            pltpu.VMEM((BANK, CHUNK), jnp.int32),
            pltpu.VMEM((NB, CHUNK, 128), jnp.float32),
            [[pltpu.SemaphoreType.DMA] * GSPLIT] * NB,
            [pltpu.SemaphoreType.DMA] * NB,
        ],
    )
    def seg(rows_hbm, src_hbm, dst_hbm, init_hbm, out_hbm,
            accum, sbank, dbank, rows_v, gsems, ssems):
        cid = lax.axis_index("c")
        tid = lax.axis_index("s")
        pltpu.sync_copy(init_hbm.at[pl.ds(cid * RPT, RPT)],
                        accum.at[pl.ds(tid * RPT, RPT)])
        plsc.subcore_barrier()

        def g_copy(c, b, h):
            idx = sbank.at[c, pl.ds(h * SUB, SUB)]
            src = (rows_hbm.at[idx] if col_fn is None
                   else rows_hbm.at[idx, pl.ds(col_fn(cid), 128)])
            return pltpu.make_async_copy(
                src, rows_v.at[b, pl.ds(h * SUB, SUB)], gsems[b][h])

        def g_start(c, b):
            for h in range(GSPLIT):
                g_copy(c, b, h).start()

        def g_wait(c, b):
            for h in range(GSPLIT):
                g_copy(c, b, h).wait()

        def s_copy(c, b):
            return pltpu.make_async_copy(
                rows_v.at[b], accum.at[dbank.at[c]], ssems[b])

        for k in range(nbanks):
            pltpu.sync_copy(
                src_hbm.at[pl.ds(src_rows_fn(cid, tid) + k * BANK, BANK)],
                sbank)
            pltpu.sync_copy(
                dst_hbm.at[pl.ds(dst_rows_fn(cid, tid) + k * BANK, BANK)],
                dbank)
            g_start(0, 0)

            def pair(p, carry):
                for b in range(NB):
                    c = p * NB + b

                    @pl.when(c + 1 < BANK)
                    def _prefetch():
                        @pl.when(c >= 1)
                        def _drain():
                            s_copy(c - 1, 1 - b).wait()

                        g_start(c + 1, 1 - b)

                    g_wait(c, b)
                    s_copy(c, b).start(add=True)
                return carry

            lax.fori_loop(0, BANK // NB, pair, 0)
            s_copy(BANK - 2, 0).wait()
            s_copy(BANK - 1, 1).wait()

        plsc.subcore_barrier()
        _copy_out(accum, out_hbm, cid, tid)

    return seg


_seg_sum_l1 = _make_seg_sum(
    NCH,
    lambda cid, tid: tid * NCH,
    lambda cid, tid: tid * NCH,
    col_fn=lambda cid: cid * 128,
)


def _edge_rows(cid, tid):
    return cid * (EPAD // (2 * CHUNK)) + tid * NCH2


_seg_sum_l2 = _make_seg_sum(NCH2, _edge_rows, _edge_rows)


def _tc_mm_body(a_ref, b_ref, w1a_ref, w1b_ref, b1_ref, w2_ref, p_ref):
    h = jnp.dot(a_ref[...], w1a_ref[...], preferred_element_type=jnp.float32)
    h = h + jnp.dot(b_ref[...], w1b_ref[...],
                    preferred_element_type=jnp.float32)
    h = jnp.maximum(h + b1_ref[...], 0.0)
    p = jnp.dot(h, w2_ref[...], preferred_element_type=jnp.float32)
    p_ref[...] = jnp.pad(p, ((0, 0), (0, 128 - DC)))


def _tc_dense(agg, W1a, W1b, b1, W2):
    BM = 2000
    return pl.pallas_call(
        _tc_mm_body,
        grid=(N // BM,),
        in_specs=[
            pl.BlockSpec((BM, 128), lambda i: (i, 0)),
            pl.BlockSpec((BM, 128), lambda i: (N // BM + i, 0)),
            pl.BlockSpec((128, DF), lambda i: (0, 0)),
            pl.BlockSpec((128, DF), lambda i: (0, 0)),
            pl.BlockSpec((1, DF), lambda i: (0, 0)),
            pl.BlockSpec((DF, DC), lambda i: (0, 0)),
        ],
        out_specs=pl.BlockSpec((BM, 128), lambda i: (i, 0)),
        out_shape=jax.ShapeDtypeStruct((N, 128), jnp.float32),
        compiler_params=pltpu.CompilerParams(
            dimension_semantics=("arbitrary",)),
    )(agg, agg, W1a, W1b, b1, W2)


def _tc_comb_body(a_ref, b_ref, o_ref):
    o_ref[...] = a_ref[:, :DC] + b_ref[:, :DC]


def _tc_combine(out2):
    BM = 2000
    return pl.pallas_call(
        _tc_comb_body,
        grid=(N // BM,),
        in_specs=[
            pl.BlockSpec((BM, 128), lambda i: (i, 0)),
            pl.BlockSpec((BM, 128), lambda i: (N // BM + i, 0)),
        ],
        out_specs=pl.BlockSpec((BM, DC), lambda i: (i, 0)),
        out_shape=jax.ShapeDtypeStruct((N, DC), jnp.float32),
        compiler_params=pltpu.CompilerParams(
            dimension_semantics=("arbitrary",)),
    )(out2, out2)


def kernel(x, adj, W1, b1, W2, b2):
    src = adj[0].astype(jnp.int32)
    dst = adj[1].astype(jnp.int32)
    pad = EPAD - E
    pad_i = jnp.arange(pad, dtype=jnp.int32)
    src_p = jnp.concatenate([src, pad_i % N])
    dst_p = jnp.concatenate([dst, N + pad_i % (RACC - N)])
    src2d = src_p.reshape(-1, CHUNK)
    dst2d = dst_p.reshape(-1, CHUNK)

    zinit = jnp.zeros((2 * RPT, 128), jnp.float32)
    agg = _seg_sum_l1(x, src2d, dst2d, zinit)

    p128 = _tc_dense(agg, W1[:128], W1[128:], b1.reshape(1, DF), W2)

    binit = jnp.concatenate([
        jnp.broadcast_to(jnp.pad(b2, (0, 128 - DC)), (RPT, 128)),
        jnp.zeros((RPT, 128), jnp.float32),
    ])
    out2 = _seg_sum_l2(p128, src2d, dst2d, binit)
    return _tc_combine(out2)


# speedup vs baseline: 1.0258x; 1.0004x over previous
import functools

import jax
import jax.numpy as jnp
from jax import lax
from jax.experimental import pallas as pl
from jax.experimental.pallas import tpu as pltpu
from jax.experimental.pallas import tpu_sc as plsc

N = 10000
E = 160000
DF = 256
DC = 64

NC = 2
NT = 16
CHUNK = 128
EPT = 10240
EPAD = NT * EPT
NCH = EPT // CHUNK
EPT2 = EPAD // (NC * NT)
NCH2 = EPT2 // CHUNK

RACC = 10112
RPT = RACC // NT
OPT = 624

NB = 2

_MESH = plsc.VectorSubcoreMesh(core_axis_name="c", subcore_axis_name="s")


def _copy_out(accum, out_hbm, cid, tid):
    pltpu.sync_copy(accum.at[pl.ds(tid * OPT, OPT)],
                    out_hbm.at[pl.ds(cid * N + tid * OPT, OPT)])

    @pl.when(tid == NT - 1)
    def _tail():
        pltpu.sync_copy(accum.at[pl.ds(NT * OPT, N - NT * OPT)],
                        out_hbm.at[pl.ds(cid * N + NT * OPT, N - NT * OPT)])


BANK = 40


GSPLIT = 4
SUB = CHUNK // GSPLIT


def _make_seg_sum(nch, src_rows_fn, dst_rows_fn, col_fn=None):
    nbanks = nch // BANK

    @functools.partial(
        pl.kernel,
        out_type=jax.ShapeDtypeStruct((2 * N, 128), jnp.float32),
        mesh=_MESH,
        scratch_types=[
            pltpu.VMEM_SHARED((RACC, 128), jnp.float32),
            pltpu.VMEM((BANK, CHUNK), jnp.int32),
            pltpu.VMEM((BANK, CHUNK), jnp.int32),
            pltpu.VMEM((NB, CHUNK, 128), jnp.float32),
            [[pltpu.SemaphoreType.DMA] * GSPLIT] * NB,
            [pltpu.SemaphoreType.DMA] * NB,
        ],
    )
    def seg(rows_hbm, src_hbm, dst_hbm, stail_hbm, dtail_hbm, init_hbm,
            out_hbm, accum, sbank, dbank, rows_v, gsems, ssems):
        cid = lax.axis_index("c")
        tid = lax.axis_index("s")
        pltpu.sync_copy(init_hbm.at[pl.ds(cid * RPT, RPT)],
                        accum.at[pl.ds(tid * RPT, RPT)])
        plsc.subcore_barrier()

        def g_copy(c, b, h):
            idx = sbank.at[c, pl.ds(h * SUB, SUB)]
            src = (rows_hbm.at[idx] if col_fn is None
                   else rows_hbm.at[idx, pl.ds(col_fn(cid), 128)])
            return pltpu.make_async_copy(
                src, rows_v.at[b, pl.ds(h * SUB, SUB)], gsems[b][h])

        def g_start(c, b):
            for h in range(GSPLIT):
                g_copy(c, b, h).start()

        def g_wait(c, b):
            for h in range(GSPLIT):
                g_copy(c, b, h).wait()

        def s_copy(c, b):
            return pltpu.make_async_copy(
                rows_v.at[b], accum.at[dbank.at[c]], ssems[b])

        for k in range(nbanks):
            row0 = src_rows_fn(cid, tid) + k * BANK

            @pl.when(row0 + BANK <= E // CHUNK)
            def _real():
                pltpu.sync_copy(src_hbm.at[pl.ds(row0, BANK)], sbank)
                pltpu.sync_copy(dst_hbm.at[pl.ds(row0, BANK)], dbank)

            @pl.when(row0 + BANK > E // CHUNK)
            def _tail_bank():
                pltpu.sync_copy(stail_hbm, sbank)
                pltpu.sync_copy(dtail_hbm, dbank)

            g_start(0, 0)

            def pair(p, carry):
                for b in range(NB):
                    c = p * NB + b

                    @pl.when(c + 1 < BANK)
                    def _prefetch():
                        @pl.when(c >= 1)
                        def _drain():
                            s_copy(c - 1, 1 - b).wait()

                        g_start(c + 1, 1 - b)

                    g_wait(c, b)
                    s_copy(c, b).start(add=True)
                return carry

            lax.fori_loop(0, BANK // NB, pair, 0)
            s_copy(BANK - 2, 0).wait()
            s_copy(BANK - 1, 1).wait()

        plsc.subcore_barrier()
        _copy_out(accum, out_hbm, cid, tid)

    return seg


_seg_sum_l1 = _make_seg_sum(
    NCH,
    lambda cid, tid: tid * NCH,
    lambda cid, tid: tid * NCH,
    col_fn=lambda cid: cid * 128,
)


def _edge_rows(cid, tid):
    return cid * (EPAD // (2 * CHUNK)) + tid * NCH2


_seg_sum_l2 = _make_seg_sum(NCH2, _edge_rows, _edge_rows)


def _tc_mm_body(a_ref, b_ref, w1a_ref, w1b_ref, b1_ref, w2_ref, p_ref):
    h = jnp.dot(a_ref[...], w1a_ref[...], preferred_element_type=jnp.float32)
    h = h + jnp.dot(b_ref[...], w1b_ref[...],
                    preferred_element_type=jnp.float32)
    h = jnp.maximum(h + b1_ref[...], 0.0)
    p = jnp.dot(h, w2_ref[...], preferred_element_type=jnp.float32)
    p_ref[...] = jnp.pad(p, ((0, 0), (0, 128 - DC)))


def _tc_dense(agg, W1a, W1b, b1, W2):
    BM = 2000
    return pl.pallas_call(
        _tc_mm_body,
        grid=(N // BM,),
        in_specs=[
            pl.BlockSpec((BM, 128), lambda i: (i, 0)),
            pl.BlockSpec((BM, 128), lambda i: (N // BM + i, 0)),
            pl.BlockSpec((128, DF), lambda i: (0, 0)),
            pl.BlockSpec((128, DF), lambda i: (0, 0)),
            pl.BlockSpec((1, DF), lambda i: (0, 0)),
            pl.BlockSpec((DF, DC), lambda i: (0, 0)),
        ],
        out_specs=pl.BlockSpec((BM, 128), lambda i: (i, 0)),
        out_shape=jax.ShapeDtypeStruct((N, 128), jnp.float32),
        compiler_params=pltpu.CompilerParams(
            dimension_semantics=("arbitrary",)),
    )(agg, agg, W1a, W1b, b1, W2)


def _tc_comb_body(a_ref, b_ref, o_ref):
    o_ref[...] = a_ref[:, :DC] + b_ref[:, :DC]


def _tc_combine(out2):
    BM = 2000
    return pl.pallas_call(
        _tc_comb_body,
        grid=(N // BM,),
        in_specs=[
            pl.BlockSpec((BM, 128), lambda i: (i, 0)),
            pl.BlockSpec((BM, 128), lambda i: (N // BM + i, 0)),
        ],
        out_specs=pl.BlockSpec((BM, DC), lambda i: (i, 0)),
        out_shape=jax.ShapeDtypeStruct((N, DC), jnp.float32),
        compiler_params=pltpu.CompilerParams(
            dimension_semantics=("arbitrary",)),
    )(out2, out2)


def kernel(x, adj, W1, b1, W2, b2):
    src = adj[0].astype(jnp.int32)
    dst = adj[1].astype(jnp.int32)
    pad = EPAD - E
    pad_i = jnp.arange(pad, dtype=jnp.int32)
    src2d = src.reshape(-1, CHUNK)
    dst2d = dst.reshape(-1, CHUNK)
    ntail = BANK * CHUNK - pad
    stail = jnp.concatenate([src[-ntail:], pad_i % N]).reshape(BANK, CHUNK)
    dtail = jnp.concatenate([dst[-ntail:],
                             N + pad_i % (RACC - N)]).reshape(BANK, CHUNK)

    zinit = jnp.zeros((2 * RPT, 128), jnp.float32)
    agg = _seg_sum_l1(x, src2d, dst2d, stail, dtail, zinit)

    p128 = _tc_dense(agg, W1[:128], W1[128:], b1.reshape(1, DF), W2)

    binit = jnp.concatenate([
        jnp.broadcast_to(jnp.pad(b2, (0, 128 - DC)), (RPT, 128)),
        jnp.zeros((RPT, 128), jnp.float32),
    ])
    out2 = _seg_sum_l2(p128, src2d, dst2d, stail, dtail, binit)
    return _tc_combine(out2)
